# baseline (speedup 1.0000x reference)
import jax
import jax.numpy as jnp
from jax.experimental import pallas as pl
from jax.experimental.pallas import tpu as pltpu


def _make_kernel(channels):
    def _fused_kernel(ca_ref,
                      x_ref, e_ref, n_ref,
                      w1_hbm,
                      b1_hbm, temb_hbm,
                      w2_hbm, b2_hbm,
                      out_ref,
                      w1_ref, b1_ref, temb_ref, w2_ref, b2_ref,
                      wsem):
        b = pl.program_id(0)
        k = pl.program_id(1)
        j = pl.program_id(0) * pl.num_programs(1) + k

        @pl.when(j == 0)
        def _():
            pltpu.make_async_copy(w1_hbm, w1_ref, wsem.at[0]).start()
            pltpu.make_async_copy(b1_hbm, b1_ref, wsem.at[1]).start()
            pltpu.make_async_copy(temb_hbm, temb_ref, wsem.at[2]).start()
            pltpu.make_async_copy(w2_hbm, w2_ref, wsem.at[3]).start()
            pltpu.make_async_copy(b2_hbm, b2_ref, wsem.at[4]).start()
            pltpu.make_async_copy(w1_hbm, w1_ref, wsem.at[0]).wait()
            pltpu.make_async_copy(b1_hbm, b1_ref, wsem.at[1]).wait()
            pltpu.make_async_copy(temb_hbm, temb_ref, wsem.at[2]).wait()
            pltpu.make_async_copy(w2_hbm, w2_ref, wsem.at[3]).wait()
            pltpu.make_async_copy(b2_hbm, b2_ref, wsem.at[4]).wait()

        c = ca_ref[b]
        s = jnp.sqrt(jnp.maximum(1.0 - c * c, 0.0))

        x = x_ref[0]
        e = e_ref[0]
        nz = n_ref[0]

        u = s * nz - c * e

        w1n = w1_ref[channels:].astype(jnp.bfloat16)
        wA = (w1_ref[:channels] + c * w1_ref[channels:]).astype(jnp.bfloat16)

        h = (jnp.dot(x.astype(jnp.bfloat16), wA,
                     preferred_element_type=jnp.float32)
             + jnp.dot(u.astype(jnp.bfloat16), w1n,
                       preferred_element_type=jnp.float32))
        h = h + (b1_ref[...] + c * temb_ref[...])
        th = jnp.tanh(0.5 * h)
        h = 0.5 * h * (1.0 + th)

        out = (jnp.dot(h.astype(jnp.bfloat16),
                       w2_ref[...].astype(jnp.bfloat16),
                       preferred_element_type=jnp.float32)
               + b2_ref[...])

        diff = nz - out
        psum = jnp.sum(diff * diff, axis=0)

        @pl.when(k == 0)
        def _():
            out_ref[0, 0] = jnp.zeros_like(psum)

        out_ref[0, 0] = out_ref[0, 0] + psum

    return _fused_kernel


def _pick_tile(dhw, cap=8192):
    if dhw % 8 != 0:
        return dhw
    t = min(dhw, cap)
    while dhw % t != 0:
        t -= 8
    return t


def kernel(x, e, noise, sqrt_alpha, w1, b1, temb, w2, b2):
    B, C, D, H, W = x.shape
    DHW = D * H * W
    HID = w1.shape[1]

    S = _pick_tile(DHW)
    n_tiles = DHW // S

    xt = jnp.transpose(x, (0, 2, 3, 4, 1)).reshape(B, DHW, C)
    et = jnp.transpose(e, (0, 2, 3, 4, 1)).reshape(B, DHW, C)
    nt = jnp.transpose(noise, (0, 2, 3, 4, 1)).reshape(B, DHW, C)

    grid_spec = pltpu.PrefetchScalarGridSpec(
        num_scalar_prefetch=1,
        grid=(B, n_tiles),
        in_specs=[
            pl.BlockSpec((1, S, C), lambda b, k, ca: (b, k, 0)),
            pl.BlockSpec((1, S, C), lambda b, k, ca: (b, k, 0)),
            pl.BlockSpec((1, S, C), lambda b, k, ca: (b, k, 0)),
            pl.BlockSpec(memory_space=pl.ANY),
            pl.BlockSpec(memory_space=pl.ANY),
            pl.BlockSpec(memory_space=pl.ANY),
            pl.BlockSpec(memory_space=pl.ANY),
            pl.BlockSpec(memory_space=pl.ANY),
        ],
        out_specs=pl.BlockSpec((1, 1, C), lambda b, k, ca: (b, 0, 0)),
        scratch_shapes=[
            pltpu.VMEM((2 * C, HID), jnp.float32),
            pltpu.VMEM((1, HID), jnp.float32),
            pltpu.VMEM((1, HID), jnp.float32),
            pltpu.VMEM((HID, C), jnp.float32),
            pltpu.VMEM((1, C), jnp.float32),
            pltpu.SemaphoreType.DMA((5,)),
        ],
    )

    partials = pl.pallas_call(
        _make_kernel(C),
        out_shape=jax.ShapeDtypeStruct((B, 1, C), jnp.float32),
        grid_spec=grid_spec,
        compiler_params=pltpu.CompilerParams(
            dimension_semantics=("arbitrary", "arbitrary"),
            vmem_limit_bytes=64 * 1024 * 1024),
    )(sqrt_alpha, xt, et, nt, w1, b1, temb, w2, b2)

    return jnp.sum(partials) / (B * C * DHW)


# speedup vs baseline: 1.0307x; 1.0307x over previous
import jax
import jax.numpy as jnp
from jax.experimental import pallas as pl
from jax.experimental.pallas import tpu as pltpu


def _make_kernel(channels):
    def _fused_kernel(ca_ref,
                      x_ref, e_ref, n_ref,
                      w1_ref,
                      b1_ref, temb_ref,
                      w2_ref, b2_ref,
                      out_ref):
        b = pl.program_id(0)
        k = pl.program_id(1)

        c = ca_ref[b]
        s = jnp.sqrt(jnp.maximum(1.0 - c * c, 0.0))

        x = x_ref[0]
        e = e_ref[0]
        nz = n_ref[0]

        u = s * nz - c * e

        w1n = w1_ref[channels:].astype(jnp.bfloat16)
        wA = (w1_ref[:channels] + c * w1_ref[channels:]).astype(jnp.bfloat16)

        h = (jnp.dot(x.astype(jnp.bfloat16), wA,
                     preferred_element_type=jnp.float32)
             + jnp.dot(u.astype(jnp.bfloat16), w1n,
                       preferred_element_type=jnp.float32))
        h = h + (b1_ref[...] + c * temb_ref[...])
        th = jnp.tanh(0.5 * h)
        h = 0.5 * h * (1.0 + th)

        out = (jnp.dot(h.astype(jnp.bfloat16),
                       w2_ref[...].astype(jnp.bfloat16),
                       preferred_element_type=jnp.float32)
               + b2_ref[...])

        diff = nz - out
        psum = jnp.sum(diff * diff, axis=0)

        @pl.when(k == 0)
        def _():
            out_ref[0, 0] = jnp.zeros_like(psum)

        out_ref[0, 0] = out_ref[0, 0] + psum

    return _fused_kernel


def _pick_tile(dhw, cap=8192):
    if dhw % 8 != 0:
        return dhw
    t = min(dhw, cap)
    while dhw % t != 0:
        t -= 8
    return t


def kernel(x, e, noise, sqrt_alpha, w1, b1, temb, w2, b2):
    B, C, D, H, W = x.shape
    DHW = D * H * W
    HID = w1.shape[1]

    S = _pick_tile(DHW)
    n_tiles = DHW // S

    xt = jnp.transpose(x, (0, 2, 3, 4, 1)).reshape(B, DHW, C)
    et = jnp.transpose(e, (0, 2, 3, 4, 1)).reshape(B, DHW, C)
    nt = jnp.transpose(noise, (0, 2, 3, 4, 1)).reshape(B, DHW, C)

    grid_spec = pltpu.PrefetchScalarGridSpec(
        num_scalar_prefetch=1,
        grid=(B, n_tiles),
        in_specs=[
            pl.BlockSpec((1, S, C), lambda b, k, ca: (b, k, 0)),
            pl.BlockSpec((1, S, C), lambda b, k, ca: (b, k, 0)),
            pl.BlockSpec((1, S, C), lambda b, k, ca: (b, k, 0)),
            pl.BlockSpec((2 * C, HID), lambda b, k, ca: (0, 0)),
            pl.BlockSpec((1, HID), lambda b, k, ca: (0, 0)),
            pl.BlockSpec((1, HID), lambda b, k, ca: (0, 0)),
            pl.BlockSpec((HID, C), lambda b, k, ca: (0, 0)),
            pl.BlockSpec((1, C), lambda b, k, ca: (0, 0)),
        ],
        out_specs=pl.BlockSpec((1, 1, C), lambda b, k, ca: (b, 0, 0)),
    )

    partials = pl.pallas_call(
        _make_kernel(C),
        out_shape=jax.ShapeDtypeStruct((B, 1, C), jnp.float32),
        grid_spec=grid_spec,
        compiler_params=pltpu.CompilerParams(
            dimension_semantics=("arbitrary", "arbitrary"),
            vmem_limit_bytes=64 * 1024 * 1024),
    )(sqrt_alpha, xt, et, nt, w1, b1, temb, w2, b2)

    return jnp.sum(partials) / (B * C * DHW)
